# baseline (speedup 1.0000x reference)
import functools

import jax
import jax.numpy as jnp
from jax import lax
from jax.experimental import pallas as pl
from jax.experimental.pallas import tpu as pltpu
from jax.experimental.pallas import tpu_sc as plsc

N = 10000
E = 160000
D = 256
ED = 16
H = 128
NUM_ITERS = 20
EPSILON = 0.1
GAMMA = 0.1

NC = 2
NS = 16
NPAD = 10240
EPT = E // NS
KCH = 128
NPT = NPAD // NS

NB = 2
NCHT = E // KCH
TMAX = (NCHT + NS - 1) // NS


def _sc_segsum_body(hflat, gsrc, dst, zeros, out, idxv, dstv, rows, acc,
                    isem, dsem, gsem, ssem):
  c = lax.axis_index("c")
  s = lax.axis_index("s")

  def _issue_idx(t, q):
    off = (s + t * NS) * KCH
    pltpu.async_copy(gsrc.at[pl.ds(c * E + off, KCH)], idxv.at[q], isem.at[q])
    pltpu.async_copy(dst.at[pl.ds(off, KCH)], dstv.at[q], dsem.at[q])

  def _live(t):
    return s + t * NS < NCHT

  def _wait_idx(q):
    pltpu.make_async_copy(
        gsrc.at[pl.ds(0, KCH)], idxv.at[q], isem.at[q]).wait()

  def _wait_dst(q):
    pltpu.make_async_copy(
        dst.at[pl.ds(0, KCH)], dstv.at[q], dsem.at[q]).wait()

  pltpu.sync_copy(zeros.at[pl.ds(s * NPT, NPT)], acc.at[pl.ds(s * NPT, NPT)])
  for b in range(NB):
    _issue_idx(b, b)
  for b in range(NB):
    _wait_idx(b)
    pltpu.async_copy(hflat.at[idxv.at[b]], rows.at[b], gsem.at[b])
  for b in range(NB):
    _issue_idx(NB + b, NB + b)
  plsc.subcore_barrier()

  @pl.loop(0, (TMAX + NB - 1) // NB)
  def _grp(g):
    for b in range(NB):
      j = g * NB + b

      @pl.when(_live(j))
      def _():
        _wait_dst(j % (2 * NB))
        pltpu.make_async_copy(
            hflat.at[idxv.at[b]], rows.at[b], gsem.at[b]).wait()
        pltpu.sync_copy(rows.at[b], acc.at[dstv.at[j % (2 * NB)]], add=True)
    for b in range(NB):
      j = g * NB + b
      nj = j + NB
      nnj = j + 2 * NB

      @pl.when(_live(nj))
      def _():
        _wait_idx(nj % (2 * NB))
        pltpu.async_copy(
            hflat.at[idxv.at[nj % (2 * NB)]], rows.at[b], gsem.at[b])

      @pl.when(_live(nnj))
      def _():
        _issue_idx(nnj, nnj % (2 * NB))

  plsc.subcore_barrier()
  pltpu.sync_copy(acc.at[pl.ds(s * NPT, NPT)], out.at[c, pl.ds(s * NPT, NPT)])


@functools.cache
def _get_sc_segsum():
  mesh = plsc.VectorSubcoreMesh(
      core_axis_name="c", subcore_axis_name="s", num_cores=NC, num_subcores=NS)
  return pl.kernel(
      _sc_segsum_body,
      out_type=jax.ShapeDtypeStruct((NC, NPAD, H), jnp.float32),
      mesh=mesh,
      scratch_types=[
          pltpu.VMEM((2 * NB, KCH), jnp.int32),
          pltpu.VMEM((2 * NB, KCH), jnp.int32),
          pltpu.VMEM((NB, KCH, H), jnp.float32),
          pltpu.VMEM_SHARED((NPAD, H), jnp.float32),
          pltpu.SemaphoreType.DMA((2 * NB,)),
          pltpu.SemaphoreType.DMA((2 * NB,)),
          pltpu.SemaphoreType.DMA((NB,)),
          pltpu.SemaphoreType.DMA((NB,)),
      ],
  )


BN = 1000


def _tc_prologue_body(x_ref, wpt_ref, bp_ref, out_ref):
  r = jnp.dot(x_ref[...], wpt_ref[...], preferred_element_type=jnp.float32)
  r = r + bp_ref[...]
  out_ref[0] = r[:, :H]
  out_ref[1] = r[:, H:]


def _tc_prologue(x, wpt, bp):
  return pl.pallas_call(
      _tc_prologue_body,
      grid=(N // BN,),
      in_specs=[
          pl.BlockSpec((BN, D), lambda i: (i, 0)),
          pl.BlockSpec((D, D), lambda i: (0, 0)),
          pl.BlockSpec((1, D), lambda i: (0, 0)),
      ],
      out_specs=pl.BlockSpec((NC, BN, H), lambda i: (0, i, 0)),
      out_shape=jax.ShapeDtypeStruct((NC, N, H), jnp.float32),
  )(x, wpt, bp)


def _tc_const_body(ea_ref, wet_ref, bc_ref, out_ref):
  ea = ea_ref[0][:, :ED]
  out_ref[...] = (
      jnp.dot(ea, wet_ref[...], preferred_element_type=jnp.float32)
      + bc_ref[...])


def _tc_const(ea2, wet, bc):
  return pl.pallas_call(
      _tc_const_body,
      grid=(N // BN,),
      in_specs=[
          pl.BlockSpec((NC, BN, H), lambda i: (0, i, 0)),
          pl.BlockSpec((ED, D), lambda i: (0, 0)),
          pl.BlockSpec((1, D), lambda i: (0, 0)),
      ],
      out_specs=pl.BlockSpec((BN, D), lambda i: (i, 0)),
      out_shape=jax.ShapeDtypeStruct((N, D), jnp.float32),
  )(ea2, wet, bc)


def _tc_update_body(h_ref, g_ref, c_ref, m_ref, out_ref):
  hh = jnp.concatenate([h_ref[0], h_ref[1], g_ref[0], g_ref[1]], axis=1)
  z = jnp.dot(hh, m_ref[...], preferred_element_type=jnp.float32) + c_ref[...]
  z = jnp.tanh(z)
  out_ref[0] = h_ref[0] + EPSILON * z[:, :H]
  out_ref[1] = h_ref[1] + EPSILON * z[:, H:]


def _tc_update(h2, agg2, cterm, m):
  return pl.pallas_call(
      _tc_update_body,
      grid=(N // BN,),
      in_specs=[
          pl.BlockSpec((NC, BN, H), lambda i: (0, i, 0)),
          pl.BlockSpec((NC, BN, H), lambda i: (0, i, 0)),
          pl.BlockSpec((BN, D), lambda i: (i, 0)),
          pl.BlockSpec((2 * D, D), lambda i: (0, 0)),
      ],
      out_specs=pl.BlockSpec((NC, BN, H), lambda i: (0, i, 0)),
      out_shape=jax.ShapeDtypeStruct((NC, N, H), jnp.float32),
  )(h2, agg2, cterm, m)


def _tc_readout_body(h_ref, w1t_ref, b1_ref, w2t_ref, b2_ref, out_ref):
  hh = jnp.concatenate([h_ref[0], h_ref[1]], axis=1)
  y = jnp.dot(hh, w1t_ref[...], preferred_element_type=jnp.float32)
  y = y + b1_ref[...]
  y = jnp.where(y >= 0, y, 0.01 * y)
  out_ref[...] = (
      jnp.dot(y, w2t_ref[...], preferred_element_type=jnp.float32)
      + b2_ref[...])


def _tc_readout(h2, w1t, b1, w2t, b2):
  return pl.pallas_call(
      _tc_readout_body,
      grid=(N // BN,),
      in_specs=[
          pl.BlockSpec((NC, BN, H), lambda i: (0, i, 0)),
          pl.BlockSpec((D, D), lambda i: (0, 0)),
          pl.BlockSpec((1, D), lambda i: (0, 0)),
          pl.BlockSpec((D, 1), lambda i: (0, 0)),
          pl.BlockSpec((1, 1), lambda i: (0, 0)),
      ],
      out_specs=pl.BlockSpec((BN, 1), lambda i: (i, 0)),
      out_shape=jax.ShapeDtypeStruct((N, 1), jnp.float32),
  )(h2, w1t, b1, w2t, b2)


@jax.jit
def kernel(x, edge_index, edge_attr, Wp, bp, Wc, bc, Wphi, Wedge, W1, b1, W2,
           b2):
  src = edge_index[0]
  dst = edge_index[1]
  gsrc = jnp.concatenate([src, src + N])
  a_mat = Wc - Wc.T - GAMMA * jnp.eye(D, dtype=jnp.float32)
  m = jnp.concatenate([a_mat.T, Wphi.T], axis=0)
  zeros = jnp.zeros((NPAD, H), jnp.float32)
  eattr_pad = jnp.pad(edge_attr, ((0, 0), (0, H - ED)))

  h2 = _tc_prologue(x, Wp.T, bp[None])
  eidx = jnp.arange(E, dtype=jnp.int32)
  ea2 = _get_sc_segsum()(eattr_pad, jnp.concatenate([eidx, eidx]), dst, zeros)
  cterm = _tc_const(ea2, Wedge.T, bc[None])

  sc_segsum = _get_sc_segsum()
  for _ in range(NUM_ITERS):
    hflat = h2.reshape(NC * N, H)
    agg2 = sc_segsum(hflat, gsrc, dst, zeros)
    h2 = _tc_update(h2, agg2, cterm, m)

  y = _tc_readout(h2, W1.T, b1[None], W2.T, b2[None])
  return y.reshape(-1)


# speedup vs baseline: 1.0192x; 1.0192x over previous
import functools

import jax
import jax.numpy as jnp
from jax import lax
from jax.experimental import pallas as pl
from jax.experimental.pallas import tpu as pltpu
from jax.experimental.pallas import tpu_sc as plsc

N = 10000
E = 160000
D = 256
ED = 16
H = 128
NUM_ITERS = 20
EPSILON = 0.1
GAMMA = 0.1

NC = 2
NS = 16
NPAD = 10240
EPT = E // NS
KCH = 128
NPT = NPAD // NS

NB = 2
NCHT = E // KCH
TMAX = (NCHT + NS - 1) // NS


def _sc_segsum_body(hflat, gsrc, dst, zeros, out, idxv, dstv, rows, acc,
                    isem, dsem, gsem, ssem):
  c = lax.axis_index("c")
  s = lax.axis_index("s")

  def _issue_idx(t, q):
    off = (s + t * NS) * KCH
    pltpu.async_copy(gsrc.at[pl.ds(c * E + off, KCH)], idxv.at[q], isem.at[q])
    pltpu.async_copy(dst.at[pl.ds(off, KCH)], dstv.at[q], dsem.at[q])

  def _live(t):
    return s + t * NS < NCHT

  def _wait_idx(q):
    pltpu.make_async_copy(
        gsrc.at[pl.ds(0, KCH)], idxv.at[q], isem.at[q]).wait()

  def _wait_dst(q):
    pltpu.make_async_copy(
        dst.at[pl.ds(0, KCH)], dstv.at[q], dsem.at[q]).wait()

  pltpu.sync_copy(zeros.at[pl.ds(s * NPT, NPT)], acc.at[pl.ds(s * NPT, NPT)])
  for b in range(NB):
    _issue_idx(b, b)
  for b in range(NB):
    _wait_idx(b)
    pltpu.async_copy(hflat.at[idxv.at[b]], rows.at[b], gsem.at[b])
  for b in range(NB):
    _issue_idx(NB + b, NB + b)
  plsc.subcore_barrier()

  @pl.loop(0, (TMAX + NB - 1) // NB)
  def _grp(g):
    for b in range(NB):
      j = g * NB + b

      @pl.when(_live(j))
      def _():
        _wait_dst(j % (2 * NB))
        pltpu.make_async_copy(
            hflat.at[idxv.at[b]], rows.at[b], gsem.at[b]).wait()
        pltpu.async_copy(
            rows.at[b], acc.at[dstv.at[j % (2 * NB)]], ssem.at[b], add=True)
    for b in range(NB):
      j = g * NB + b
      nj = j + NB
      nnj = j + 2 * NB

      @pl.when(_live(j))
      def _():
        pltpu.make_async_copy(
            rows.at[b], acc.at[dstv.at[j % (2 * NB)]], ssem.at[b]).wait()

      @pl.when(_live(nj))
      def _():
        _wait_idx(nj % (2 * NB))
        pltpu.async_copy(
            hflat.at[idxv.at[nj % (2 * NB)]], rows.at[b], gsem.at[b])

      @pl.when(_live(nnj))
      def _():
        _issue_idx(nnj, nnj % (2 * NB))

  plsc.subcore_barrier()
  pltpu.sync_copy(acc.at[pl.ds(s * NPT, NPT)], out.at[c, pl.ds(s * NPT, NPT)])


@functools.cache
def _get_sc_segsum():
  mesh = plsc.VectorSubcoreMesh(
      core_axis_name="c", subcore_axis_name="s", num_cores=NC, num_subcores=NS)
  return pl.kernel(
      _sc_segsum_body,
      out_type=jax.ShapeDtypeStruct((NC, NPAD, H), jnp.float32),
      mesh=mesh,
      scratch_types=[
          pltpu.VMEM((2 * NB, KCH), jnp.int32),
          pltpu.VMEM((2 * NB, KCH), jnp.int32),
          pltpu.VMEM((NB, KCH, H), jnp.float32),
          pltpu.VMEM_SHARED((NPAD, H), jnp.float32),
          pltpu.SemaphoreType.DMA((2 * NB,)),
          pltpu.SemaphoreType.DMA((2 * NB,)),
          pltpu.SemaphoreType.DMA((NB,)),
          pltpu.SemaphoreType.DMA((NB,)),
      ],
  )


BN = 1000


def _tc_prologue_body(x_ref, wpt_ref, bp_ref, out_ref):
  r = jnp.dot(x_ref[...], wpt_ref[...], preferred_element_type=jnp.float32)
  r = r + bp_ref[...]
  out_ref[0] = r[:, :H]
  out_ref[1] = r[:, H:]


def _tc_prologue(x, wpt, bp):
  return pl.pallas_call(
      _tc_prologue_body,
      grid=(N // BN,),
      in_specs=[
          pl.BlockSpec((BN, D), lambda i: (i, 0)),
          pl.BlockSpec((D, D), lambda i: (0, 0)),
          pl.BlockSpec((1, D), lambda i: (0, 0)),
      ],
      out_specs=pl.BlockSpec((NC, BN, H), lambda i: (0, i, 0)),
      out_shape=jax.ShapeDtypeStruct((NC, N, H), jnp.float32),
  )(x, wpt, bp)


def _tc_const_body(ea_ref, wet_ref, bc_ref, out_ref):
  ea = ea_ref[0][:, :ED]
  out_ref[...] = (
      jnp.dot(ea, wet_ref[...], preferred_element_type=jnp.float32)
      + bc_ref[...])


def _tc_const(ea2, wet, bc):
  return pl.pallas_call(
      _tc_const_body,
      grid=(N // BN,),
      in_specs=[
          pl.BlockSpec((NC, BN, H), lambda i: (0, i, 0)),
          pl.BlockSpec((ED, D), lambda i: (0, 0)),
          pl.BlockSpec((1, D), lambda i: (0, 0)),
      ],
      out_specs=pl.BlockSpec((BN, D), lambda i: (i, 0)),
      out_shape=jax.ShapeDtypeStruct((N, D), jnp.float32),
  )(ea2, wet, bc)


def _tc_update_body(h_ref, g_ref, c_ref, m_ref, out_ref):
  hh = jnp.concatenate([h_ref[0], h_ref[1], g_ref[0], g_ref[1]], axis=1)
  z = jnp.dot(hh, m_ref[...], preferred_element_type=jnp.float32) + c_ref[...]
  z = jnp.tanh(z)
  out_ref[0] = h_ref[0] + EPSILON * z[:, :H]
  out_ref[1] = h_ref[1] + EPSILON * z[:, H:]


def _tc_update(h2, agg2, cterm, m):
  return pl.pallas_call(
      _tc_update_body,
      grid=(N // BN,),
      in_specs=[
          pl.BlockSpec((NC, BN, H), lambda i: (0, i, 0)),
          pl.BlockSpec((NC, BN, H), lambda i: (0, i, 0)),
          pl.BlockSpec((BN, D), lambda i: (i, 0)),
          pl.BlockSpec((2 * D, D), lambda i: (0, 0)),
      ],
      out_specs=pl.BlockSpec((NC, BN, H), lambda i: (0, i, 0)),
      out_shape=jax.ShapeDtypeStruct((NC, N, H), jnp.float32),
  )(h2, agg2, cterm, m)


def _tc_readout_body(h_ref, w1t_ref, b1_ref, w2t_ref, b2_ref, out_ref):
  hh = jnp.concatenate([h_ref[0], h_ref[1]], axis=1)
  y = jnp.dot(hh, w1t_ref[...], preferred_element_type=jnp.float32)
  y = y + b1_ref[...]
  y = jnp.where(y >= 0, y, 0.01 * y)
  out_ref[...] = (
      jnp.dot(y, w2t_ref[...], preferred_element_type=jnp.float32)
      + b2_ref[...])


def _tc_readout(h2, w1t, b1, w2t, b2):
  return pl.pallas_call(
      _tc_readout_body,
      grid=(N // BN,),
      in_specs=[
          pl.BlockSpec((NC, BN, H), lambda i: (0, i, 0)),
          pl.BlockSpec((D, D), lambda i: (0, 0)),
          pl.BlockSpec((1, D), lambda i: (0, 0)),
          pl.BlockSpec((D, 1), lambda i: (0, 0)),
          pl.BlockSpec((1, 1), lambda i: (0, 0)),
      ],
      out_specs=pl.BlockSpec((BN, 1), lambda i: (i, 0)),
      out_shape=jax.ShapeDtypeStruct((N, 1), jnp.float32),
  )(h2, w1t, b1, w2t, b2)


@jax.jit
def kernel(x, edge_index, edge_attr, Wp, bp, Wc, bc, Wphi, Wedge, W1, b1, W2,
           b2):
  src = edge_index[0]
  dst = edge_index[1]
  gsrc = jnp.concatenate([src, src + N])
  a_mat = Wc - Wc.T - GAMMA * jnp.eye(D, dtype=jnp.float32)
  m = jnp.concatenate([a_mat.T, Wphi.T], axis=0)
  zeros = jnp.zeros((NPAD, H), jnp.float32)
  eattr_pad = jnp.pad(edge_attr, ((0, 0), (0, H - ED)))

  h2 = _tc_prologue(x, Wp.T, bp[None])
  eidx = jnp.arange(E, dtype=jnp.int32)
  ea2 = _get_sc_segsum()(eattr_pad, jnp.concatenate([eidx, eidx]), dst, zeros)
  cterm = _tc_const(ea2, Wedge.T, bc[None])

  sc_segsum = _get_sc_segsum()
  for _ in range(NUM_ITERS):
    hflat = h2.reshape(NC * N, H)
    agg2 = sc_segsum(hflat, gsrc, dst, zeros)
    h2 = _tc_update(h2, agg2, cterm, m)

  y = _tc_readout(h2, W1.T, b1[None], W2.T, b2[None])
  return y.reshape(-1)


# speedup vs baseline: 1.2898x; 1.2655x over previous
import functools

import jax
import jax.numpy as jnp
from jax import lax
from jax.experimental import pallas as pl
from jax.experimental.pallas import tpu as pltpu
from jax.experimental.pallas import tpu_sc as plsc

N = 10000
E = 160000
D = 256
ED = 16
H = 128
NUM_ITERS = 20
EPSILON = 0.1
GAMMA = 0.1

NC = 2
NS = 16
NPAD = 10240
EPT = E // NS
KCH = 80
NPT = NPAD // NS

NB = 4
NCHUNK = EPT // KCH


def _sc_segsum_body(hflat, gsrc, dst, zeros, out, idxv, dstv, rows, acc,
                    isem, dsem, gsem, ssem):
  c = lax.axis_index("c")
  s = lax.axis_index("s")

  def _issue_idx(t, q):
    off = s * EPT + t * KCH
    pltpu.async_copy(gsrc.at[pl.ds(c * E + off, KCH)], idxv.at[q], isem.at[q])
    pltpu.async_copy(dst.at[pl.ds(off, KCH)], dstv.at[q], dsem.at[q])

  def _live(t):
    return t < NCHUNK

  def _wait_idx(q):
    pltpu.make_async_copy(
        gsrc.at[pl.ds(0, KCH)], idxv.at[q], isem.at[q]).wait()

  def _wait_dst(q):
    pltpu.make_async_copy(
        dst.at[pl.ds(0, KCH)], dstv.at[q], dsem.at[q]).wait()

  pltpu.sync_copy(zeros.at[pl.ds(s * NPT, NPT)], acc.at[pl.ds(s * NPT, NPT)])
  for b in range(NB):
    _issue_idx(b, b)
  for b in range(NB):
    _wait_idx(b)
    pltpu.async_copy(hflat.at[idxv.at[b]], rows.at[b], gsem.at[b])
  for b in range(NB):
    _issue_idx(NB + b, NB + b)
  plsc.subcore_barrier()

  @pl.loop(0, (NCHUNK + NB - 1) // NB)
  def _grp(g):
    for b in range(NB):
      j = g * NB + b

      @pl.when(_live(j))
      def _():
        _wait_dst(j % (2 * NB))
        pltpu.make_async_copy(
            hflat.at[idxv.at[b]], rows.at[b], gsem.at[b]).wait()
        pltpu.async_copy(
            rows.at[b], acc.at[dstv.at[j % (2 * NB)]], ssem.at[b], add=True)
    for b in range(NB):
      j = g * NB + b
      nj = j + NB
      nnj = j + 2 * NB

      @pl.when(_live(j))
      def _():
        pltpu.make_async_copy(
            rows.at[b], acc.at[dstv.at[j % (2 * NB)]], ssem.at[b]).wait()

      @pl.when(_live(nj))
      def _():
        _wait_idx(nj % (2 * NB))
        pltpu.async_copy(
            hflat.at[idxv.at[nj % (2 * NB)]], rows.at[b], gsem.at[b])

      @pl.when(_live(nnj))
      def _():
        _issue_idx(nnj, nnj % (2 * NB))

  plsc.subcore_barrier()
  pltpu.sync_copy(acc.at[pl.ds(s * NPT, NPT)], out.at[c, pl.ds(s * NPT, NPT)])


@functools.cache
def _get_sc_segsum():
  mesh = plsc.VectorSubcoreMesh(
      core_axis_name="c", subcore_axis_name="s", num_cores=NC, num_subcores=NS)
  return pl.kernel(
      _sc_segsum_body,
      out_type=jax.ShapeDtypeStruct((NC, NPAD, H), jnp.float32),
      mesh=mesh,
      scratch_types=[
          pltpu.VMEM((2 * NB, KCH), jnp.int32),
          pltpu.VMEM((2 * NB, KCH), jnp.int32),
          pltpu.VMEM((NB, KCH, H), jnp.float32),
          pltpu.VMEM_SHARED((NPAD, H), jnp.float32),
          pltpu.SemaphoreType.DMA((2 * NB,)),
          pltpu.SemaphoreType.DMA((2 * NB,)),
          pltpu.SemaphoreType.DMA((NB,)),
          pltpu.SemaphoreType.DMA((NB,)),
      ],
  )


BN = 1000


def _tc_prologue_body(x_ref, wpt_ref, bp_ref, out_ref):
  r = jnp.dot(x_ref[...], wpt_ref[...], preferred_element_type=jnp.float32)
  r = r + bp_ref[...]
  out_ref[0] = r[:, :H]
  out_ref[1] = r[:, H:]


def _tc_prologue(x, wpt, bp):
  return pl.pallas_call(
      _tc_prologue_body,
      grid=(N // BN,),
      in_specs=[
          pl.BlockSpec((BN, D), lambda i: (i, 0)),
          pl.BlockSpec((D, D), lambda i: (0, 0)),
          pl.BlockSpec((1, D), lambda i: (0, 0)),
      ],
      out_specs=pl.BlockSpec((NC, BN, H), lambda i: (0, i, 0)),
      out_shape=jax.ShapeDtypeStruct((NC, N, H), jnp.float32),
  )(x, wpt, bp)


def _tc_const_body(ea_ref, wet_ref, bc_ref, out_ref):
  ea = ea_ref[0][:, :ED]
  out_ref[...] = (
      jnp.dot(ea, wet_ref[...], preferred_element_type=jnp.float32)
      + bc_ref[...])


def _tc_const(ea2, wet, bc):
  return pl.pallas_call(
      _tc_const_body,
      grid=(N // BN,),
      in_specs=[
          pl.BlockSpec((NC, BN, H), lambda i: (0, i, 0)),
          pl.BlockSpec((ED, D), lambda i: (0, 0)),
          pl.BlockSpec((1, D), lambda i: (0, 0)),
      ],
      out_specs=pl.BlockSpec((BN, D), lambda i: (i, 0)),
      out_shape=jax.ShapeDtypeStruct((N, D), jnp.float32),
  )(ea2, wet, bc)


def _tc_update_body(h_ref, g_ref, c_ref, m_ref, out_ref):
  hh = jnp.concatenate([h_ref[0], h_ref[1], g_ref[0], g_ref[1]], axis=1)
  z = jnp.dot(hh, m_ref[...], preferred_element_type=jnp.float32) + c_ref[...]
  z = jnp.tanh(z)
  out_ref[0] = h_ref[0] + EPSILON * z[:, :H]
  out_ref[1] = h_ref[1] + EPSILON * z[:, H:]


def _tc_update(h2, agg2, cterm, m):
  return pl.pallas_call(
      _tc_update_body,
      grid=(N // BN,),
      in_specs=[
          pl.BlockSpec((NC, BN, H), lambda i: (0, i, 0)),
          pl.BlockSpec((NC, BN, H), lambda i: (0, i, 0)),
          pl.BlockSpec((BN, D), lambda i: (i, 0)),
          pl.BlockSpec((2 * D, D), lambda i: (0, 0)),
      ],
      out_specs=pl.BlockSpec((NC, BN, H), lambda i: (0, i, 0)),
      out_shape=jax.ShapeDtypeStruct((NC, N, H), jnp.float32),
  )(h2, agg2, cterm, m)


def _tc_readout_body(h_ref, w1t_ref, b1_ref, w2t_ref, b2_ref, out_ref):
  hh = jnp.concatenate([h_ref[0], h_ref[1]], axis=1)
  y = jnp.dot(hh, w1t_ref[...], preferred_element_type=jnp.float32)
  y = y + b1_ref[...]
  y = jnp.where(y >= 0, y, 0.01 * y)
  out_ref[...] = (
      jnp.dot(y, w2t_ref[...], preferred_element_type=jnp.float32)
      + b2_ref[...])


def _tc_readout(h2, w1t, b1, w2t, b2):
  return pl.pallas_call(
      _tc_readout_body,
      grid=(N // BN,),
      in_specs=[
          pl.BlockSpec((NC, BN, H), lambda i: (0, i, 0)),
          pl.BlockSpec((D, D), lambda i: (0, 0)),
          pl.BlockSpec((1, D), lambda i: (0, 0)),
          pl.BlockSpec((D, 1), lambda i: (0, 0)),
          pl.BlockSpec((1, 1), lambda i: (0, 0)),
      ],
      out_specs=pl.BlockSpec((BN, 1), lambda i: (i, 0)),
      out_shape=jax.ShapeDtypeStruct((N, 1), jnp.float32),
  )(h2, w1t, b1, w2t, b2)


@jax.jit
def kernel(x, edge_index, edge_attr, Wp, bp, Wc, bc, Wphi, Wedge, W1, b1, W2,
           b2):
  src = edge_index[0]
  dst = edge_index[1]
  gsrc = jnp.concatenate([src, src + N])
  a_mat = Wc - Wc.T - GAMMA * jnp.eye(D, dtype=jnp.float32)
  m = jnp.concatenate([a_mat.T, Wphi.T], axis=0)
  zeros = jnp.zeros((NPAD, H), jnp.float32)
  eattr_pad = jnp.pad(edge_attr, ((0, 0), (0, H - ED)))

  h2 = _tc_prologue(x, Wp.T, bp[None])
  eidx = jnp.arange(E, dtype=jnp.int32)
  ea2 = _get_sc_segsum()(eattr_pad, jnp.concatenate([eidx, eidx]), dst, zeros)
  cterm = _tc_const(ea2, Wedge.T, bc[None])

  sc_segsum = _get_sc_segsum()
  for _ in range(NUM_ITERS):
    hflat = h2.reshape(NC * N, H)
    agg2 = sc_segsum(hflat, gsrc, dst, zeros)
    h2 = _tc_update(h2, agg2, cterm, m)

  y = _tc_readout(h2, W1.T, b1[None], W2.T, b2[None])
  return y.reshape(-1)
